# baseline (speedup 1.0000x reference)
import functools

import jax
import jax.numpy as jnp
from jax import lax
from jax.experimental import pallas as pl
from jax.experimental.pallas import tpu as pltpu
from jax.experimental.pallas import tpu_sc as plsc

_B, _S, _V, _D, _L = 4096, 200, 100000, 64, 10
_NC, _NS = 2, 16
_NW = _NC * _NS
_BPW = _B // _NW
_SEGS = ((0, 104), (104, 96))
_NCH = len(_SEGS)
_VR = _D // 16
_NBUF = 6


def _bag_body(x_hbm, emb_hbm, doc_hbm, idx_v, rows_v, doc_v, sem):
    wid = lax.axis_index("s") * _NC + lax.axis_index("c")
    pltpu.sync_copy(x_hbm.at[pl.ds(wid * _BPW, _BPW)], idx_v)

    def fire_chunk(r, j, p):
        off, ln = _SEGS[j]
        pltpu.async_copy(
            emb_hbm.at[idx_v.at[r, pl.ds(off, ln)]],
            rows_v.at[p, pl.ds(off, ln)],
            sem.at[p, j],
        )

    def wait_chunk(j, p):
        off, ln = _SEGS[j]
        pltpu.make_async_copy(
            emb_hbm.at[idx_v.at[0, pl.ds(0, ln)]],
            rows_v.at[p, pl.ds(off, ln)],
            sem.at[p, j],
        ).wait()

    def row_phase(r, p):
        acc = [jnp.zeros((16,), jnp.float32) for _ in range(_VR)]
        for j in range(_NCH):
            wait_chunk(j, p)

            @pl.when(r + _NBUF - 1 < _BPW)
            def _():
                fire_chunk(r + _NBUF - 1, j, (p + _NBUF - 1) % _NBUF)

            def s_body(s, a):
                return [
                    a[c] + rows_v[p, s, c * 16:(c + 1) * 16]
                    for c in range(_VR)
                ]

            off, ln = _SEGS[j]
            acc = lax.fori_loop(off, off + ln, s_body, acc, unroll=8)
        for c in range(_VR):
            doc_v[r, c * 16:(c + 1) * 16] = acc[c]

    for r in range(_NBUF - 1):
        for j in range(_NCH):
            fire_chunk(r, j, r)

    def group_body(i, carry):
        for p in range(_NBUF):
            row_phase(_NBUF * i + p, p)
        return carry

    n_groups = _BPW // _NBUF
    lax.fori_loop(0, n_groups, group_body, 0)
    for r in range(n_groups * _NBUF, _BPW):
        row_phase(r, r % _NBUF)
    pltpu.sync_copy(doc_v, doc_hbm.at[pl.ds(wid * _BPW, _BPW)])


@functools.partial(jax.jit, static_argnums=())
def _bag(x_flat, emb):
    mesh = plsc.VectorSubcoreMesh(core_axis_name="c", subcore_axis_name="s")
    return pl.kernel(
        _bag_body,
        out_type=jax.ShapeDtypeStruct((_B, _D), jnp.float32),
        mesh=mesh,
        scratch_types=[
            pltpu.VMEM((_BPW, _S), jnp.int32),
            pltpu.VMEM((_NBUF, _S, _D), jnp.float32),
            pltpu.VMEM((_BPW, _D), jnp.float32),
            pltpu.SemaphoreType.DMA((_NBUF, _NCH)),
        ],
        compiler_params=pltpu.CompilerParams(use_tc_tiling_on_sc=False),
    )(x_flat, emb)


def _head_body(doc_ref, w_ref, b_ref, out_ref):
    bin_doc = (doc_ref[...] > 0.0).astype(jnp.float32)
    res = lax.dot_general(
        bin_doc, w_ref[...], (((1,), (1,)), ((), ())),
        preferred_element_type=jnp.float32,
    )
    out_ref[...] = res + b_ref[...][None, :]


def _head(doc, w, b):
    blk = 1024
    return pl.pallas_call(
        _head_body,
        out_shape=jax.ShapeDtypeStruct((_B, _L), jnp.float32),
        grid=(_B // blk,),
        in_specs=[
            pl.BlockSpec((blk, _D), lambda i: (i, 0)),
            pl.BlockSpec((_L, _D), lambda i: (0, 0)),
            pl.BlockSpec((_L,), lambda i: (0,)),
        ],
        out_specs=pl.BlockSpec((blk, _L), lambda i: (i, 0)),
    )(doc, w, b)


def kernel(x, m, emb, W, b):
    del m
    doc = _bag(x.astype(jnp.int32), emb)
    out = _head(doc, W, b)
    return (out,)


# speedup vs baseline: 1.0385x; 1.0385x over previous
import functools

import jax
import jax.numpy as jnp
from jax import lax
from jax.experimental import pallas as pl
from jax.experimental.pallas import tpu as pltpu
from jax.experimental.pallas import tpu_sc as plsc

_B, _S, _V, _D, _L = 4096, 200, 100000, 64, 10
_NC, _NS = 2, 16
_NW = _NC * _NS
_BPW = _B // _NW
_SEGS = ((0, 104), (104, 96))
_NCH = len(_SEGS)
_VR = _D // 16
_NBUF = 4


def _bag_body(x_hbm, emb_hbm, doc_hbm, idx_v, rows_v, doc_v, sem):
    wid = lax.axis_index("s") * _NC + lax.axis_index("c")
    pltpu.sync_copy(x_hbm.at[pl.ds(wid * _BPW, _BPW)], idx_v)

    def fire_chunk(r, j, p):
        off, ln = _SEGS[j]
        pltpu.async_copy(
            emb_hbm.at[idx_v.at[r, pl.ds(off, ln)]],
            rows_v.at[p, pl.ds(off, ln)],
            sem.at[p, j],
        )

    def wait_chunk(j, p):
        off, ln = _SEGS[j]
        pltpu.make_async_copy(
            emb_hbm.at[idx_v.at[0, pl.ds(0, ln)]],
            rows_v.at[p, pl.ds(off, ln)],
            sem.at[p, j],
        ).wait()

    def row_phase(r, p):
        acc = [jnp.zeros((16,), jnp.float32) for _ in range(_VR)]
        for j in range(_NCH):
            wait_chunk(j, p)

            @pl.when(r + _NBUF - 1 < _BPW)
            def _():
                fire_chunk(r + _NBUF - 1, j, (p + _NBUF - 1) % _NBUF)

            def s_body(s, a):
                return [
                    a[c] + rows_v[p, s, c * 16:(c + 1) * 16]
                    for c in range(_VR)
                ]

            off, ln = _SEGS[j]
            acc = lax.fori_loop(off, off + ln, s_body, acc, unroll=8)
        for c in range(_VR):
            doc_v[r, c * 16:(c + 1) * 16] = acc[c]

    for r in range(_NBUF - 1):
        for j in range(_NCH):
            fire_chunk(r, j, r)

    def group_body(i, carry):
        for p in range(_NBUF):
            row_phase(_NBUF * i + p, p)
        return carry

    n_groups = _BPW // _NBUF
    lax.fori_loop(0, n_groups, group_body, 0)
    for r in range(n_groups * _NBUF, _BPW):
        row_phase(r, r % _NBUF)
    pltpu.sync_copy(doc_v, doc_hbm.at[pl.ds(wid * _BPW, _BPW)])


@functools.partial(jax.jit, static_argnums=())
def _bag(x_flat, emb):
    mesh = plsc.VectorSubcoreMesh(core_axis_name="c", subcore_axis_name="s")
    return pl.kernel(
        _bag_body,
        out_type=jax.ShapeDtypeStruct((_B, _D), jnp.float32),
        mesh=mesh,
        scratch_types=[
            pltpu.VMEM((_BPW, _S), jnp.int32),
            pltpu.VMEM((_NBUF, _S, _D), jnp.float32),
            pltpu.VMEM((_BPW, _D), jnp.float32),
            pltpu.SemaphoreType.DMA((_NBUF, _NCH)),
        ],
        compiler_params=pltpu.CompilerParams(use_tc_tiling_on_sc=False),
    )(x_flat, emb)


def _head_body(doc_ref, w_ref, b_ref, out_ref):
    bin_doc = (doc_ref[...] > 0.0).astype(jnp.float32)
    res = lax.dot_general(
        bin_doc, w_ref[...], (((1,), (1,)), ((), ())),
        preferred_element_type=jnp.float32,
    )
    out_ref[...] = res + b_ref[...][None, :]


def _head(doc, w, b):
    blk = 1024
    return pl.pallas_call(
        _head_body,
        out_shape=jax.ShapeDtypeStruct((_B, _L), jnp.float32),
        grid=(_B // blk,),
        in_specs=[
            pl.BlockSpec((blk, _D), lambda i: (i, 0)),
            pl.BlockSpec((_L, _D), lambda i: (0, 0)),
            pl.BlockSpec((_L,), lambda i: (0,)),
        ],
        out_specs=pl.BlockSpec((blk, _L), lambda i: (i, 0)),
    )(doc, w, b)


def kernel(x, m, emb, W, b):
    del m
    doc = _bag(x.astype(jnp.int32), emb)
    out = _head(doc, W, b)
    return (out,)


# speedup vs baseline: 1.0484x; 1.0096x over previous
import functools

import jax
import jax.numpy as jnp
from jax import lax
from jax.experimental import pallas as pl
from jax.experimental.pallas import tpu as pltpu
from jax.experimental.pallas import tpu_sc as plsc

_B, _S, _V, _D, _L = 4096, 200, 100000, 64, 10
_NC, _NS = 2, 16
_NW = _NC * _NS
_BPW = _B // _NW
_SEGS = ((0, 104), (104, 96))
_NCH = len(_SEGS)
_VR = _D // 16
_NBUF = 4


def _bag_body(x_hbm, emb_hbm, doc_hbm, idx_v, rows_v, doc_v, sem):
    wid = lax.axis_index("s") * _NC + lax.axis_index("c")
    pltpu.sync_copy(
        x_hbm.at[pl.ds(wid * _BPW, _NBUF)], idx_v.at[pl.ds(0, _NBUF)]
    )

    def fire_chunk(r, j, p):
        off, ln = _SEGS[j]
        pltpu.async_copy(
            emb_hbm.at[idx_v.at[r, pl.ds(off, ln)]],
            rows_v.at[p, pl.ds(off, ln)],
            sem.at[p, j],
        )

    def wait_chunk(j, p):
        off, ln = _SEGS[j]
        pltpu.make_async_copy(
            emb_hbm.at[idx_v.at[0, pl.ds(0, ln)]],
            rows_v.at[p, pl.ds(off, ln)],
            sem.at[p, j],
        ).wait()

    def row_phase(r, p):
        acc = [jnp.zeros((16,), jnp.float32) for _ in range(_VR)]
        for j in range(_NCH):
            wait_chunk(j, p)

            @pl.when(r + _NBUF - 1 < _BPW)
            def _():
                fire_chunk(r + _NBUF - 1, j, (p + _NBUF - 1) % _NBUF)

            def s_body(s, a):
                return [
                    a[c] + rows_v[p, s, c * 16:(c + 1) * 16]
                    for c in range(_VR)
                ]

            off, ln = _SEGS[j]
            acc = lax.fori_loop(off, off + ln, s_body, acc, unroll=ln // 8)
        for c in range(_VR):
            doc_v[r, c * 16:(c + 1) * 16] = acc[c]

    for r in range(_NBUF - 1):
        for j in range(_NCH):
            fire_chunk(r, j, r)

    pltpu.sync_copy(
        x_hbm.at[pl.ds(wid * _BPW + _NBUF, _BPW - _NBUF)],
        idx_v.at[pl.ds(_NBUF, _BPW - _NBUF)],
    )

    def group_body(i, carry):
        for p in range(_NBUF):
            row_phase(_NBUF * i + p, p)
        return carry

    n_groups = _BPW // _NBUF
    lax.fori_loop(0, n_groups, group_body, 0)
    for r in range(n_groups * _NBUF, _BPW):
        row_phase(r, r % _NBUF)
    pltpu.sync_copy(doc_v, doc_hbm.at[pl.ds(wid * _BPW, _BPW)])


@functools.partial(jax.jit, static_argnums=())
def _bag(x_flat, emb):
    mesh = plsc.VectorSubcoreMesh(core_axis_name="c", subcore_axis_name="s")
    return pl.kernel(
        _bag_body,
        out_type=jax.ShapeDtypeStruct((_B, _D), jnp.float32),
        mesh=mesh,
        scratch_types=[
            pltpu.VMEM((_BPW, _S), jnp.int32),
            pltpu.VMEM((_NBUF, _S, _D), jnp.float32),
            pltpu.VMEM((_BPW, _D), jnp.float32),
            pltpu.SemaphoreType.DMA((_NBUF, _NCH)),
        ],
        compiler_params=pltpu.CompilerParams(use_tc_tiling_on_sc=False),
    )(x_flat, emb)


def _head_body(doc_ref, w_ref, b_ref, out_ref):
    bin_doc = (doc_ref[...] > 0.0).astype(jnp.float32)
    res = lax.dot_general(
        bin_doc, w_ref[...], (((1,), (1,)), ((), ())),
        preferred_element_type=jnp.float32,
    )
    out_ref[...] = res + b_ref[...][None, :]


def _head(doc, w, b):
    blk = 1024
    return pl.pallas_call(
        _head_body,
        out_shape=jax.ShapeDtypeStruct((_B, _L), jnp.float32),
        grid=(_B // blk,),
        in_specs=[
            pl.BlockSpec((blk, _D), lambda i: (i, 0)),
            pl.BlockSpec((_L, _D), lambda i: (0, 0)),
            pl.BlockSpec((_L,), lambda i: (0,)),
        ],
        out_specs=pl.BlockSpec((blk, _L), lambda i: (i, 0)),
    )(doc, w, b)


def kernel(x, m, emb, W, b):
    del m
    doc = _bag(x.astype(jnp.int32), emb)
    out = _head(doc, W, b)
    return (out,)
